# baseline (speedup 1.0000x reference)
import functools

import jax
import jax.numpy as jnp
from jax import lax
from jax.experimental import pallas as pl
from jax.experimental.pallas import tpu as pltpu
from jax.experimental.pallas import tpu_sc as plsc

TOTAL = 32768
D = 512
S = 16
NEG = -1e30

NC = 2
NS = 16
NW = NC * NS

SC_ROWS = 2048
TC_ROWS = TOTAL - SC_ROWS
BLK1 = 3840
NBLK1 = TC_ROWS // BLK1
BLK2 = 4096
NBLK2 = TOTAL // BLK2
ROWS_PER_TILE = SC_ROWS // NW
CHUNK = 64
NCH = ROWS_PER_TILE // CHUNK


def _sc_segsum_body(
    x_hbm, seg_hbm, zacc_hbm, zcnt_hbm,
    psum_hbm, pcnt_hbm,
    buf, idxv, acc_l, cnt_l, sem,
):
    c = lax.axis_index("c")
    s = lax.axis_index("s")
    wid = s * NC + c
    row0 = TC_ROWS + wid * ROWS_PER_TILE

    pltpu.sync_copy(zacc_hbm, acc_l)
    pltpu.sync_copy(zcnt_hbm, cnt_l)

    ones16 = jnp.ones((16,), jnp.float32)
    full16 = jnp.full((16,), 16.0, jnp.float32)

    pltpu.async_copy(x_hbm.at[pl.ds(row0, CHUNK)], buf.at[0], sem.at[0])

    def chunk_body(k, carry):
        p = lax.rem(k, 2)
        base = row0 + k * CHUNK

        @pl.when(k + 1 < NCH)
        def _():
            pltpu.async_copy(
                x_hbm.at[pl.ds(base + CHUNK, CHUNK)], buf.at[1 - p], sem.at[1 - p]
            )

        pltpu.make_async_copy(
            x_hbm.at[pl.ds(base, CHUNK)], buf.at[p], sem.at[p]
        ).wait()
        pltpu.sync_copy(seg_hbm.at[pl.ds(base, CHUNK)], idxv.at[pl.ds(0, CHUNK)])

        def grp_body(g, c2):
            gbase = g * 16
            seg_vec = idxv[pl.ds(gbase, 16)]
            t0 = seg_vec[0]
            tl = seg_vec[15]

            def uniform():
                accs = [buf[p, gbase, pl.ds(j * 16, 16)] for j in range(D // 16)]
                for rr in range(1, 16):
                    accs = [
                        accs[j] + buf[p, gbase + rr, pl.ds(j * 16, 16)]
                        for j in range(D // 16)
                    ]
                for j in range(D // 16):
                    sl = pl.ds(j * 16, 16)
                    acc_l[t0, sl] = acc_l[t0, sl] + accs[j]
                cnt_l[t0, :] = cnt_l[t0, :] + full16

            def mixed():
                def row_body(rr, c3):
                    t = idxv[pl.ds(gbase + rr, 16)][0]
                    for j in range(D // 16):
                        sl = pl.ds(j * 16, 16)
                        acc_l[t, sl] = acc_l[t, sl] + buf[p, gbase + rr, sl]
                    cnt_l[t, :] = cnt_l[t, :] + ones16
                    return c3

                lax.fori_loop(0, 16, row_body, 0)

            lax.cond(t0 == tl, uniform, mixed)
            return c2

        lax.fori_loop(0, CHUNK // 16, grp_body, 0)
        return carry

    lax.fori_loop(0, NCH, chunk_body, 0)

    pltpu.sync_copy(acc_l, psum_hbm.at[wid])
    pltpu.sync_copy(cnt_l, pcnt_hbm.at[wid])


def _tc_segsum_kernel(x_ref, seg_ref, acc_out, cnt_out, acc_ref, cnt_ref):
    i = pl.program_id(0)

    @pl.when(i == 0)
    def _():
        acc_ref[...] = jnp.zeros_like(acc_ref)
        cnt_ref[...] = jnp.zeros_like(cnt_ref)

    seg = seg_ref[0, 0, :]
    oh = (seg[:, None] == jax.lax.broadcasted_iota(jnp.int32, (BLK1, S), 1)).astype(
        jnp.float32
    )
    x = x_ref[...]
    acc_ref[...] += jax.lax.dot_general(oh, x, (((0,), (0,)), ((), ())))
    cnt_ref[...] += jnp.sum(oh, axis=0, keepdims=True)

    @pl.when(i == NBLK1 - 1)
    def _():
        acc_out[...] = acc_ref[...]
        cnt_out[...] = cnt_ref[...]


def _attn_kernel(
    x_ref, seg_ref, psum_ref, pcnt_ref, tacc_ref, tcnt_ref, wq_ref, wk_ref,
    wo_ref, out_ref, qk_ref, m_ref, d_ref, z_ref,
):
    i = pl.program_id(0)

    @pl.when(i == 0)
    def _():
        acc = jnp.sum(psum_ref[...], axis=0) + tacc_ref[...]
        cnt = jnp.maximum(jnp.sum(pcnt_ref[...], axis=0)[:, 0] + tcnt_ref[0, :], 1.0)
        mean = acc / cnt[:, None]
        q = jnp.dot(mean, wq_ref[...])
        qk_ref[...] = lax.dot_general(q, wk_ref[...], (((1,), (1,)), ((), ())))
        m_ref[...] = jnp.full_like(m_ref, NEG)
        d_ref[...] = jnp.zeros_like(d_ref)
        z_ref[...] = jnp.zeros_like(z_ref)

    seg = seg_ref[0, 0, :]
    ohb = seg[:, None] == jax.lax.broadcasted_iota(jnp.int32, (BLK2, S), 1)
    x = x_ref[...]
    P = jax.lax.dot_general(x, qk_ref[...], (((1,), (1,)), ((), ())))
    Pm = jnp.where(ohb, P, NEG)

    t = Pm
    while t.shape[0] > 8:
        h = t.shape[0] // 2
        t = jnp.maximum(t[:h], t[h:])
    bm = jnp.max(t, axis=0)
    m_old = m_ref[0, :]
    m_new = jnp.maximum(m_old, bm)
    scale = jnp.exp(m_old - m_new)
    m_safe = jnp.where(m_new == NEG, 0.0, m_new)
    W = jnp.exp(Pm - m_safe[None, :])
    u = W
    while u.shape[0] > 8:
        h = u.shape[0] // 2
        u = u[:h] + u[h:]
    d_ref[0, :] = d_ref[0, :] * scale + jnp.sum(u, axis=0)
    z_ref[...] = z_ref[...] * scale[:, None] + jax.lax.dot_general(
        W, x, (((0,), (0,)), ((), ()))
    )
    m_ref[0, :] = m_new

    @pl.when(i == NBLK2 - 1)
    def _():
        d = d_ref[0, :]
        dd = jnp.where(d > 0, d, 1.0)
        out_ref[...] = jnp.dot(z_ref[...] / dd[:, None], wo_ref[...])


def kernel(element_embeddings, element_to_sample_map, num_samples, Wq, Wk, Wo):
    x = element_embeddings
    seg_i32 = element_to_sample_map.astype(jnp.int32)
    seg31 = seg_i32[:TC_ROWS].reshape(NBLK1, 1, BLK1)
    seg32 = seg_i32.reshape(NBLK2, 1, BLK2)

    mesh = plsc.VectorSubcoreMesh(
        core_axis_name="c", subcore_axis_name="s", num_cores=NC, num_subcores=NS
    )
    sc_segsum = functools.partial(
        pl.kernel,
        mesh=mesh,
        out_type=[
            jax.ShapeDtypeStruct((NW, S, D), jnp.float32),
            jax.ShapeDtypeStruct((NW, S, 16), jnp.float32),
        ],
        scratch_types=[
            pltpu.VMEM((2, CHUNK, D), jnp.float32),
            pltpu.VMEM((CHUNK + 16,), jnp.int32),
            pltpu.VMEM((S, D), jnp.float32),
            pltpu.VMEM((S, 16), jnp.float32),
            pltpu.SemaphoreType.DMA((2,)),
        ],
    )(_sc_segsum_body)

    zacc = jnp.zeros((S, D), jnp.float32)
    zcnt = jnp.zeros((S, 16), jnp.float32)
    psum, pcnt = sc_segsum(x, seg_i32, zacc, zcnt)

    tacc, tcnt = pl.pallas_call(
        _tc_segsum_kernel,
        grid=(NBLK1,),
        in_specs=[
            pl.BlockSpec((BLK1, D), lambda i: (i, 0)),
            pl.BlockSpec((1, 1, BLK1), lambda i: (i, 0, 0)),
        ],
        out_specs=[
            pl.BlockSpec((S, D), lambda i: (0, 0)),
            pl.BlockSpec((1, S), lambda i: (0, 0)),
        ],
        out_shape=[
            jax.ShapeDtypeStruct((S, D), jnp.float32),
            jax.ShapeDtypeStruct((1, S), jnp.float32),
        ],
        scratch_shapes=[
            pltpu.VMEM((S, D), jnp.float32),
            pltpu.VMEM((1, S), jnp.float32),
        ],
    )(x, seg31)

    out = pl.pallas_call(
        _attn_kernel,
        grid=(NBLK2,),
        in_specs=[
            pl.BlockSpec((BLK2, D), lambda i: (i, 0)),
            pl.BlockSpec((1, 1, BLK2), lambda i: (i, 0, 0)),
            pl.BlockSpec((NW, S, D), lambda i: (0, 0, 0)),
            pl.BlockSpec((NW, S, 16), lambda i: (0, 0, 0)),
            pl.BlockSpec((S, D), lambda i: (0, 0)),
            pl.BlockSpec((1, S), lambda i: (0, 0)),
            pl.BlockSpec((D, D), lambda i: (0, 0)),
            pl.BlockSpec((D, D), lambda i: (0, 0)),
            pl.BlockSpec((D, D), lambda i: (0, 0)),
        ],
        out_specs=pl.BlockSpec((S, D), lambda i: (0, 0)),
        out_shape=jax.ShapeDtypeStruct((S, D), jnp.float32),
        scratch_shapes=[
            pltpu.VMEM((S, D), jnp.float32),
            pltpu.VMEM((1, S), jnp.float32),
            pltpu.VMEM((1, S), jnp.float32),
            pltpu.VMEM((S, D), jnp.float32),
        ],
    )(x, seg32, psum, pcnt, tacc, tcnt, Wq, Wk, Wo)
    return out


# speedup vs baseline: 1.0145x; 1.0145x over previous
import functools

import jax
import jax.numpy as jnp
from jax import lax
from jax.experimental import pallas as pl
from jax.experimental.pallas import tpu as pltpu
from jax.experimental.pallas import tpu_sc as plsc

TOTAL = 32768
D = 512
S = 16
NEG = -1e30

NC = 2
NS = 16
NW = NC * NS

SC_ROWS = 2048
TC_ROWS = TOTAL - SC_ROWS
BLK1 = 3840
NBLK1 = TC_ROWS // BLK1
BLK2 = 4096
NBLK2 = TOTAL // BLK2
ROWS_PER_TILE = SC_ROWS // NW
CHUNK = 64
NCH = ROWS_PER_TILE // CHUNK


def _sc_segsum_body(
    x_hbm, seg_hbm, zacc_hbm, zcnt_hbm,
    psum_hbm, pcnt_hbm,
    buf, idxv, acc_l, cnt_l, sem,
):
    c = lax.axis_index("c")
    s = lax.axis_index("s")
    wid = s * NC + c
    row0 = TC_ROWS + wid * ROWS_PER_TILE

    pltpu.sync_copy(zacc_hbm, acc_l)
    pltpu.sync_copy(zcnt_hbm, cnt_l)

    ones16 = jnp.ones((16,), jnp.float32)
    full16 = jnp.full((16,), 16.0, jnp.float32)

    pltpu.async_copy(x_hbm.at[pl.ds(row0, CHUNK)], buf.at[0], sem.at[0])

    def chunk_body(k, carry):
        p = lax.rem(k, 2)
        base = row0 + k * CHUNK

        @pl.when(k + 1 < NCH)
        def _():
            pltpu.async_copy(
                x_hbm.at[pl.ds(base + CHUNK, CHUNK)], buf.at[1 - p], sem.at[1 - p]
            )

        pltpu.make_async_copy(
            x_hbm.at[pl.ds(base, CHUNK)], buf.at[p], sem.at[p]
        ).wait()
        pltpu.sync_copy(seg_hbm.at[pl.ds(base, CHUNK)], idxv.at[pl.ds(0, CHUNK)])

        def grp_body(g, c2):
            gbase = g * 16
            seg_vec = idxv[pl.ds(gbase, 16)]
            t0 = seg_vec[0]
            tl = seg_vec[15]

            def uniform():
                accs = [buf[p, gbase, pl.ds(j * 16, 16)] for j in range(D // 16)]
                for rr in range(1, 16):
                    accs = [
                        accs[j] + buf[p, gbase + rr, pl.ds(j * 16, 16)]
                        for j in range(D // 16)
                    ]
                for j in range(D // 16):
                    sl = pl.ds(j * 16, 16)
                    acc_l[t0, sl] = acc_l[t0, sl] + accs[j]
                cnt_l[t0, :] = cnt_l[t0, :] + full16

            def mixed():
                def row_body(rr, c3):
                    t = idxv[pl.ds(gbase + rr, 16)][0]
                    for j in range(D // 16):
                        sl = pl.ds(j * 16, 16)
                        acc_l[t, sl] = acc_l[t, sl] + buf[p, gbase + rr, sl]
                    cnt_l[t, :] = cnt_l[t, :] + ones16
                    return c3

                lax.fori_loop(0, 16, row_body, 0)

            lax.cond(t0 == tl, uniform, mixed)
            return c2

        lax.fori_loop(0, CHUNK // 16, grp_body, 0)
        return carry

    lax.fori_loop(0, NCH, chunk_body, 0)

    pltpu.sync_copy(acc_l, psum_hbm.at[wid])
    pltpu.sync_copy(cnt_l, pcnt_hbm.at[wid])


def _tc_segsum_kernel(x_ref, seg_ref, acc_out, cnt_out, acc_ref, cnt_ref):
    i = pl.program_id(0)

    @pl.when(i == 0)
    def _():
        acc_ref[...] = jnp.zeros_like(acc_ref)
        cnt_ref[...] = jnp.zeros_like(cnt_ref)

    seg = seg_ref[0, 0, :]
    oh = (seg[:, None] == jax.lax.broadcasted_iota(jnp.int32, (BLK1, S), 1)).astype(
        jnp.float32
    )
    x = x_ref[...]
    acc_ref[...] += jax.lax.dot_general(oh, x, (((0,), (0,)), ((), ())))
    cnt_ref[...] += jnp.sum(oh, axis=0, keepdims=True)

    @pl.when(i == NBLK1 - 1)
    def _():
        acc_out[...] = acc_ref[...]
        cnt_out[...] = cnt_ref[...]


def _attn_kernel(
    x_ref, seg_ref, psum_ref, pcnt_ref, tacc_ref, tcnt_ref, wq_ref, wk_ref,
    wo_ref, out_ref, qk_ref, m_ref, d_ref, z_ref,
):
    i = pl.program_id(0)

    @pl.when(i == 0)
    def _():
        acc = jnp.sum(psum_ref[...], axis=0) + tacc_ref[...]
        cnt = jnp.maximum(jnp.sum(pcnt_ref[...], axis=0)[:, 0] + tcnt_ref[0, :], 1.0)
        mean = acc / cnt[:, None]
        q = jnp.dot(mean, wq_ref[...])
        qk_ref[...] = lax.dot_general(q, wk_ref[...], (((1,), (1,)), ((), ())))
        m_ref[...] = jnp.full_like(m_ref, NEG)
        d_ref[...] = jnp.zeros_like(d_ref)
        z_ref[...] = jnp.zeros_like(z_ref)

    seg = seg_ref[0, 0, :]
    ohb = seg[:, None] == jax.lax.broadcasted_iota(jnp.int32, (BLK2, S), 1)
    x = x_ref[...]
    P = jax.lax.dot_general(x, qk_ref[...], (((1,), (1,)), ((), ())))
    Pm = jnp.where(ohb, P, NEG)
    bm = jnp.max(Pm, axis=0)
    m_old = m_ref[0, :]
    m_new = jnp.maximum(m_old, bm)
    scale = jnp.exp(m_old - m_new)
    m_safe = jnp.where(m_new == NEG, 0.0, m_new)
    W = jnp.exp(Pm - m_safe[None, :])
    d_ref[0, :] = d_ref[0, :] * scale + jnp.sum(W, axis=0)
    z_ref[...] = z_ref[...] * scale[:, None] + jax.lax.dot_general(
        W, x, (((0,), (0,)), ((), ()))
    )
    m_ref[0, :] = m_new

    @pl.when(i == NBLK2 - 1)
    def _():
        d = d_ref[0, :]
        dd = jnp.where(d > 0, d, 1.0)
        out_ref[...] = jnp.dot(z_ref[...] / dd[:, None], wo_ref[...])


def kernel(element_embeddings, element_to_sample_map, num_samples, Wq, Wk, Wo):
    x = element_embeddings
    seg_i32 = element_to_sample_map.astype(jnp.int32)
    seg31 = seg_i32[:TC_ROWS].reshape(NBLK1, 1, BLK1)
    seg32 = seg_i32.reshape(NBLK2, 1, BLK2)

    mesh = plsc.VectorSubcoreMesh(
        core_axis_name="c", subcore_axis_name="s", num_cores=NC, num_subcores=NS
    )
    sc_segsum = functools.partial(
        pl.kernel,
        mesh=mesh,
        out_type=[
            jax.ShapeDtypeStruct((NW, S, D), jnp.float32),
            jax.ShapeDtypeStruct((NW, S, 16), jnp.float32),
        ],
        scratch_types=[
            pltpu.VMEM((2, CHUNK, D), jnp.float32),
            pltpu.VMEM((CHUNK + 16,), jnp.int32),
            pltpu.VMEM((S, D), jnp.float32),
            pltpu.VMEM((S, 16), jnp.float32),
            pltpu.SemaphoreType.DMA((2,)),
        ],
    )(_sc_segsum_body)

    zacc = jnp.zeros((S, D), jnp.float32)
    zcnt = jnp.zeros((S, 16), jnp.float32)
    psum, pcnt = sc_segsum(x, seg_i32, zacc, zcnt)

    tacc, tcnt = pl.pallas_call(
        _tc_segsum_kernel,
        grid=(NBLK1,),
        in_specs=[
            pl.BlockSpec((BLK1, D), lambda i: (i, 0)),
            pl.BlockSpec((1, 1, BLK1), lambda i: (i, 0, 0)),
        ],
        out_specs=[
            pl.BlockSpec((S, D), lambda i: (0, 0)),
            pl.BlockSpec((1, S), lambda i: (0, 0)),
        ],
        out_shape=[
            jax.ShapeDtypeStruct((S, D), jnp.float32),
            jax.ShapeDtypeStruct((1, S), jnp.float32),
        ],
        scratch_shapes=[
            pltpu.VMEM((S, D), jnp.float32),
            pltpu.VMEM((1, S), jnp.float32),
        ],
    )(x, seg31)

    out = pl.pallas_call(
        _attn_kernel,
        grid=(NBLK2,),
        in_specs=[
            pl.BlockSpec((BLK2, D), lambda i: (i, 0)),
            pl.BlockSpec((1, 1, BLK2), lambda i: (i, 0, 0)),
            pl.BlockSpec((NW, S, D), lambda i: (0, 0, 0)),
            pl.BlockSpec((NW, S, 16), lambda i: (0, 0, 0)),
            pl.BlockSpec((S, D), lambda i: (0, 0)),
            pl.BlockSpec((1, S), lambda i: (0, 0)),
            pl.BlockSpec((D, D), lambda i: (0, 0)),
            pl.BlockSpec((D, D), lambda i: (0, 0)),
            pl.BlockSpec((D, D), lambda i: (0, 0)),
        ],
        out_specs=pl.BlockSpec((S, D), lambda i: (0, 0)),
        out_shape=jax.ShapeDtypeStruct((S, D), jnp.float32),
        scratch_shapes=[
            pltpu.VMEM((S, D), jnp.float32),
            pltpu.VMEM((1, S), jnp.float32),
            pltpu.VMEM((1, S), jnp.float32),
            pltpu.VMEM((S, D), jnp.float32),
        ],
    )(x, seg32, psum, pcnt, tacc, tcnt, Wq, Wk, Wo)
    return out
